# baseline (speedup 1.0000x reference)
import functools

import jax
import jax.numpy as jnp
from jax import lax
from jax.experimental import pallas as pl
from jax.experimental.pallas import tpu as pltpu
from jax.experimental.pallas import tpu_sc as plsc

N_EXP = 64
K = 8
TOKENS = 8192
NW = 32
CHUNKS = 2
CTOK = TOKENS // CHUNKS
WPC = NW // CHUNKS
TPW = TOKENS // NW
NG = TPW // 16
NCHAIN = 4


def _scores_body(x_ref, w_ref, s_ref):
    x = x_ref[...]
    w = w_ref[...]
    logits = jax.lax.dot_general(
        w, x, (((1,), (1,)), ((), ())), preferred_element_type=jnp.float32
    )
    m = jnp.max(logits, axis=0, keepdims=True)
    ex = jnp.exp(logits - m)
    s_ref[...] = ex / jnp.sum(ex, axis=0, keepdims=True)


def _routing_body(s0_hbm, s1_hbm, idx_hbm, wgt_hbm, cnt_hbm,
                  s_v, valt_v, idx_v, wgt_v, cnt_v, sem):
    wid = lax.axis_index("s") * 2 + lax.axis_index("c")
    base = wid * TPW
    cbase = lax.rem(wid, WPC) * TPW

    @pl.when(wid < WPC)
    def _():
        pltpu.sync_copy(s0_hbm.at[:, pl.ds(cbase, TPW)], s_v)

    @pl.when(wid >= WPC)
    def _():
        pltpu.sync_copy(s1_hbm.at[:, pl.ds(cbase, TPW)], s_v)

    zero16 = jnp.zeros((16,), jnp.int32)
    neg = jnp.full((16,), -jnp.inf, jnp.float32)
    ones = jnp.ones((16,), jnp.int32)
    lane = lax.iota(jnp.int32, 16)
    estep = N_EXP // NCHAIN

    def round_body(r, _):
        rvec = jnp.full((16,), r, jnp.int32)

        @plsc.parallel_loop(0, NG)
        def _grp(g):
            tok = lane + g * 16
            g16 = g * 16
            ms = [neg] * NCHAIN
            bis = [zero16] * NCHAIN
            for j in range(estep):
                for c in range(NCHAIN):
                    e = c * estep + j
                    v = s_v[e, pl.ds(g16, 16)]
                    gt = v > ms[c]
                    ms[c] = jnp.where(gt, v, ms[c])
                    bis[c] = jnp.where(gt, jnp.full((16,), e, jnp.int32), bis[c])
            while len(ms) > 1:
                nm, nb = [], []
                for c in range(0, len(ms), 2):
                    a_wins = (ms[c] > ms[c + 1]) | (
                        (ms[c] == ms[c + 1]) & (bis[c] < bis[c + 1])
                    )
                    nm.append(jnp.where(a_wins, ms[c], ms[c + 1]))
                    nb.append(jnp.where(a_wins, bis[c], bis[c + 1]))
                ms, bis = nm, nb
            m, bi = ms[0], bis[0]
            plsc.store_scatter(idx_v, [tok * K + rvec], bi)
            valt_v[pl.ds(r * TPW + g * 16, 16)] = m
            plsc.store_scatter(s_v, [bi, tok], neg)

        return 0

    lax.fori_loop(0, K, round_body, 0)

    for j in range(N_EXP):
        cnt_v[pl.ds(j * 16, 16)] = zero16
    for j in range(TPW * K // 16):
        iv = idx_v[pl.ds(j * 16, 16)]
        plsc.addupdate_scatter(cnt_v, [iv * 16 + lane], ones)

    for g in range(NG):
        tok = lane + g * 16
        tot = jnp.zeros((16,), jnp.float32)
        for r in range(K):
            tot = tot + valt_v[pl.ds(r * TPW + g * 16, 16)]
        tot = tot + 1e-20
        for r in range(K):
            w = valt_v[pl.ds(r * TPW + g * 16, 16)] / tot
            plsc.store_scatter(wgt_v, [tok * K + r], w)

    pltpu.sync_copy(idx_v, idx_hbm.at[pl.ds(base * K, TPW * K)])
    pltpu.sync_copy(wgt_v, wgt_hbm.at[pl.ds(base * K, TPW * K)])
    pltpu.sync_copy(cnt_v, cnt_hbm.at[wid])


def kernel(hidden_states, weight):
    bsz, seq, d = hidden_states.shape
    tokens = bsz * seq
    x = hidden_states.reshape(tokens, d)
    bt = 1024
    bpc = CTOK // bt

    score_chunks = []
    for c in range(CHUNKS):
        score_chunks.append(pl.pallas_call(
            _scores_body,
            grid=(bpc,),
            in_specs=[
                pl.BlockSpec((bt, d), lambda i, c=c: (c * bpc + i, 0)),
                pl.BlockSpec((N_EXP, d), lambda i: (0, 0)),
            ],
            out_specs=pl.BlockSpec((N_EXP, bt), lambda i: (0, i)),
            out_shape=jax.ShapeDtypeStruct((N_EXP, CTOK), jnp.float32),
        )(x, weight))

    mesh = plsc.VectorSubcoreMesh(core_axis_name="c", subcore_axis_name="s")
    routing = functools.partial(
        pl.kernel,
        mesh=mesh,
        compiler_params=pltpu.CompilerParams(needs_layout_passes=False),
        out_type=[
            jax.ShapeDtypeStruct((tokens * K,), jnp.int32),
            jax.ShapeDtypeStruct((tokens * K,), jnp.float32),
            jax.ShapeDtypeStruct((NW, N_EXP * 16), jnp.int32),
        ],
        scratch_types=[
            pltpu.VMEM((N_EXP, TPW), jnp.float32),
            pltpu.VMEM((K * TPW,), jnp.float32),
            pltpu.VMEM((TPW * K,), jnp.int32),
            pltpu.VMEM((TPW * K,), jnp.float32),
            pltpu.VMEM((N_EXP * 16,), jnp.int32),
            pltpu.SemaphoreType.DMA,
        ],
    )(_routing_body)
    topk_idx, topk_wgt, cnt_parts = routing(*score_chunks)
    counts = jnp.sum(cnt_parts.reshape(NW, N_EXP, 16), axis=(0, 2))
    return topk_idx.reshape(tokens, K), topk_wgt.reshape(tokens, K), counts


# speedup vs baseline: 1.0546x; 1.0546x over previous
import functools

import jax
import jax.numpy as jnp
from jax import lax
from jax.experimental import pallas as pl
from jax.experimental.pallas import tpu as pltpu
from jax.experimental.pallas import tpu_sc as plsc

N_EXP = 64
K = 8
TOKENS = 8192
NW = 32
CHUNKS = 2
CTOK = TOKENS // CHUNKS
TPW = CTOK // NW
NG = TPW // 16
NCHAIN = 4


def _scores_body(x_ref, w_ref, s_ref):
    x = x_ref[...]
    w = w_ref[...]
    logits = jax.lax.dot_general(
        w, x, (((1,), (1,)), ((), ())), preferred_element_type=jnp.float32
    )
    m = jnp.max(logits, axis=0, keepdims=True)
    ex = jnp.exp(logits - m)
    s_ref[...] = ex / jnp.sum(ex, axis=0, keepdims=True)


def _routing_body(s_hbm, idx_hbm, wgt_hbm, cnt_hbm,
                  s_v, valt_v, idx_v, wgt_v, cnt_v, sem):
    wid = lax.axis_index("s") * 2 + lax.axis_index("c")
    base = wid * TPW
    pltpu.sync_copy(s_hbm.at[:, pl.ds(base, TPW)], s_v)

    zero16 = jnp.zeros((16,), jnp.int32)
    neg = jnp.full((16,), -jnp.inf, jnp.float32)
    ones = jnp.ones((16,), jnp.int32)
    lane = lax.iota(jnp.int32, 16)
    estep = N_EXP // NCHAIN

    def round_body(r, _):
        rvec = jnp.full((16,), r, jnp.int32)

        @plsc.parallel_loop(0, NG, unroll=2)
        def _grp(g):
            tok = lane + g * 16
            g16 = g * 16
            ms = [neg] * NCHAIN
            bis = [zero16] * NCHAIN
            for j in range(estep):
                for c in range(NCHAIN):
                    e = c * estep + j
                    v = s_v[e, pl.ds(g16, 16)]
                    gt = v > ms[c]
                    ms[c] = jnp.where(gt, v, ms[c])
                    bis[c] = jnp.where(gt, jnp.full((16,), e, jnp.int32), bis[c])
            while len(ms) > 1:
                nm, nb = [], []
                for c in range(0, len(ms), 2):
                    a_wins = (ms[c] > ms[c + 1]) | (
                        (ms[c] == ms[c + 1]) & (bis[c] < bis[c + 1])
                    )
                    nm.append(jnp.where(a_wins, ms[c], ms[c + 1]))
                    nb.append(jnp.where(a_wins, bis[c], bis[c + 1]))
                ms, bis = nm, nb
            m, bi = ms[0], bis[0]
            plsc.store_scatter(idx_v, [tok * K + rvec], bi)
            valt_v[pl.ds(r * TPW + g * 16, 16)] = m
            plsc.store_scatter(s_v, [bi, tok], neg)

        return 0

    lax.fori_loop(0, K, round_body, 0)

    for j in range(N_EXP):
        cnt_v[pl.ds(j * 16, 16)] = zero16
    for j in range(TPW * K // 16):
        iv = idx_v[pl.ds(j * 16, 16)]
        plsc.addupdate_scatter(cnt_v, [iv * 16 + lane], ones)

    for g in range(NG):
        tok = lane + g * 16
        tot = jnp.zeros((16,), jnp.float32)
        for r in range(K):
            tot = tot + valt_v[pl.ds(r * TPW + g * 16, 16)]
        tot = tot + 1e-20
        for r in range(K):
            w = valt_v[pl.ds(r * TPW + g * 16, 16)] / tot
            plsc.store_scatter(wgt_v, [tok * K + r], w)

    pltpu.sync_copy(idx_v, idx_hbm.at[pl.ds(base * K, TPW * K)])
    pltpu.sync_copy(wgt_v, wgt_hbm.at[pl.ds(base * K, TPW * K)])
    pltpu.sync_copy(cnt_v, cnt_hbm.at[wid])


def kernel(hidden_states, weight):
    bsz, seq, d = hidden_states.shape
    tokens = bsz * seq
    x = hidden_states.reshape(tokens, d)
    bt = 1024
    bpc = CTOK // bt

    mesh = plsc.VectorSubcoreMesh(core_axis_name="c", subcore_axis_name="s")
    routing = functools.partial(
        pl.kernel,
        mesh=mesh,
        compiler_params=pltpu.CompilerParams(needs_layout_passes=False),
        out_type=[
            jax.ShapeDtypeStruct((CTOK * K,), jnp.int32),
            jax.ShapeDtypeStruct((CTOK * K,), jnp.float32),
            jax.ShapeDtypeStruct((NW, N_EXP * 16), jnp.int32),
        ],
        scratch_types=[
            pltpu.VMEM((N_EXP, TPW), jnp.float32),
            pltpu.VMEM((K * TPW,), jnp.float32),
            pltpu.VMEM((TPW * K,), jnp.int32),
            pltpu.VMEM((TPW * K,), jnp.float32),
            pltpu.VMEM((N_EXP * 16,), jnp.int32),
            pltpu.SemaphoreType.DMA,
        ],
    )(_routing_body)

    idx_parts, wgt_parts, cnt_parts = [], [], []
    for c in range(CHUNKS):
        scores_c = pl.pallas_call(
            _scores_body,
            grid=(bpc,),
            in_specs=[
                pl.BlockSpec((bt, d), lambda i, c=c: (c * bpc + i, 0)),
                pl.BlockSpec((N_EXP, d), lambda i: (0, 0)),
            ],
            out_specs=pl.BlockSpec((N_EXP, bt), lambda i: (0, i)),
            out_shape=jax.ShapeDtypeStruct((N_EXP, CTOK), jnp.float32),
        )(x, weight)
        idx_c, wgt_c, cnt_c = routing(scores_c)
        idx_parts.append(idx_c.reshape(CTOK, K))
        wgt_parts.append(wgt_c.reshape(CTOK, K))
        cnt_parts.append(cnt_c)

    counts = jnp.sum(
        jnp.stack(cnt_parts).reshape(CHUNKS * NW, N_EXP, 16), axis=(0, 2)
    )
    topk_idx = jnp.concatenate(idx_parts, axis=0)
    topk_wgt = jnp.concatenate(wgt_parts, axis=0)
    return topk_idx, topk_wgt, counts


# speedup vs baseline: 1.0659x; 1.0107x over previous
import functools

import jax
import jax.numpy as jnp
from jax import lax
from jax.experimental import pallas as pl
from jax.experimental.pallas import tpu as pltpu
from jax.experimental.pallas import tpu_sc as plsc

N_EXP = 64
K = 8
TOKENS = 8192
NW = 32
CHUNKS = 1
CTOK = TOKENS // CHUNKS
TPW = CTOK // NW
NG = TPW // 16
NCHAIN = 4


def _scores_body(x_ref, w_ref, s_ref):
    x = x_ref[...]
    w = w_ref[...]
    logits = jax.lax.dot_general(
        w, x, (((1,), (1,)), ((), ())), preferred_element_type=jnp.float32
    )
    m = jnp.max(logits, axis=0, keepdims=True)
    ex = jnp.exp(logits - m)
    s_ref[...] = ex / jnp.sum(ex, axis=0, keepdims=True)


def _routing_body(s_hbm, idx_hbm, wgt_hbm, cnt_hbm,
                  s_v, valt_v, idx_v, wgt_v, cnt_v, sem):
    wid = lax.axis_index("s") * 2 + lax.axis_index("c")
    base = wid * TPW
    pltpu.sync_copy(s_hbm.at[:, pl.ds(base, TPW)], s_v)

    zero16 = jnp.zeros((16,), jnp.int32)
    neg = jnp.full((16,), -jnp.inf, jnp.float32)
    ones = jnp.ones((16,), jnp.int32)
    lane = lax.iota(jnp.int32, 16)
    estep = N_EXP // NCHAIN

    def round_body(r, _):
        rvec = jnp.full((16,), r, jnp.int32)

        @plsc.parallel_loop(0, NG)
        def _grp(g):
            tok = lane + g * 16
            g16 = g * 16
            ms = [neg] * NCHAIN
            bis = [zero16] * NCHAIN
            for j in range(estep):
                for c in range(NCHAIN):
                    e = c * estep + j
                    v = s_v[e, pl.ds(g16, 16)]
                    gt = v > ms[c]
                    ms[c] = jnp.where(gt, v, ms[c])
                    bis[c] = jnp.where(gt, jnp.full((16,), e, jnp.int32), bis[c])
            while len(ms) > 1:
                nm, nb = [], []
                for c in range(0, len(ms), 2):
                    a_wins = (ms[c] > ms[c + 1]) | (
                        (ms[c] == ms[c + 1]) & (bis[c] < bis[c + 1])
                    )
                    nm.append(jnp.where(a_wins, ms[c], ms[c + 1]))
                    nb.append(jnp.where(a_wins, bis[c], bis[c + 1]))
                ms, bis = nm, nb
            m, bi = ms[0], bis[0]
            plsc.store_scatter(idx_v, [tok * K + rvec], bi)
            valt_v[pl.ds(r * TPW + g * 16, 16)] = m
            plsc.store_scatter(s_v, [bi, tok], neg)

        return 0

    lax.fori_loop(0, K, round_body, 0)

    for j in range(N_EXP):
        cnt_v[pl.ds(j * 16, 16)] = zero16
    for j in range(TPW * K // 16):
        iv = idx_v[pl.ds(j * 16, 16)]
        plsc.addupdate_scatter(cnt_v, [iv * 16 + lane], ones)

    for g in range(NG):
        tok = lane + g * 16
        tot = jnp.zeros((16,), jnp.float32)
        for r in range(K):
            tot = tot + valt_v[pl.ds(r * TPW + g * 16, 16)]
        tot = tot + 1e-20
        for r in range(K):
            w = valt_v[pl.ds(r * TPW + g * 16, 16)] / tot
            plsc.store_scatter(wgt_v, [tok * K + r], w)

    pltpu.sync_copy(idx_v, idx_hbm.at[pl.ds(base * K, TPW * K)])
    pltpu.sync_copy(wgt_v, wgt_hbm.at[pl.ds(base * K, TPW * K)])
    pltpu.sync_copy(cnt_v, cnt_hbm.at[wid])


def kernel(hidden_states, weight):
    bsz, seq, d = hidden_states.shape
    tokens = bsz * seq
    x = hidden_states.reshape(tokens, d)
    bt = 1024
    bpc = CTOK // bt

    mesh = plsc.VectorSubcoreMesh(core_axis_name="c", subcore_axis_name="s")
    routing = functools.partial(
        pl.kernel,
        mesh=mesh,
        compiler_params=pltpu.CompilerParams(needs_layout_passes=False),
        out_type=[
            jax.ShapeDtypeStruct((CTOK * K,), jnp.int32),
            jax.ShapeDtypeStruct((CTOK * K,), jnp.float32),
            jax.ShapeDtypeStruct((NW, N_EXP * 16), jnp.int32),
        ],
        scratch_types=[
            pltpu.VMEM((N_EXP, TPW), jnp.float32),
            pltpu.VMEM((K * TPW,), jnp.float32),
            pltpu.VMEM((TPW * K,), jnp.int32),
            pltpu.VMEM((TPW * K,), jnp.float32),
            pltpu.VMEM((N_EXP * 16,), jnp.int32),
            pltpu.SemaphoreType.DMA,
        ],
    )(_routing_body)

    idx_parts, wgt_parts, cnt_parts = [], [], []
    for c in range(CHUNKS):
        scores_c = pl.pallas_call(
            _scores_body,
            grid=(bpc,),
            in_specs=[
                pl.BlockSpec((bt, d), lambda i, c=c: (c * bpc + i, 0)),
                pl.BlockSpec((N_EXP, d), lambda i: (0, 0)),
            ],
            out_specs=pl.BlockSpec((N_EXP, bt), lambda i: (0, i)),
            out_shape=jax.ShapeDtypeStruct((N_EXP, CTOK), jnp.float32),
        )(x, weight)
        idx_c, wgt_c, cnt_c = routing(scores_c)
        idx_parts.append(idx_c.reshape(CTOK, K))
        wgt_parts.append(wgt_c.reshape(CTOK, K))
        cnt_parts.append(cnt_c)

    counts = jnp.sum(
        jnp.stack(cnt_parts).reshape(CHUNKS * NW, N_EXP, 16), axis=(0, 2)
    )
    topk_idx = jnp.concatenate(idx_parts, axis=0)
    topk_wgt = jnp.concatenate(wgt_parts, axis=0)
    return topk_idx, topk_wgt, counts


# speedup vs baseline: 1.0678x; 1.0018x over previous
import functools

import jax
import jax.numpy as jnp
from jax import lax
from jax.experimental import pallas as pl
from jax.experimental.pallas import tpu as pltpu
from jax.experimental.pallas import tpu_sc as plsc

N_EXP = 64
K = 8
TOKENS = 8192
NW = 32
CHUNKS = 1
CTOK = TOKENS // CHUNKS
TPW = CTOK // NW
NG = TPW // 16
NCHAIN = 4


def _scores_body(x_ref, w_ref, s_ref):
    x = x_ref[...]
    w = w_ref[...]
    logits = jax.lax.dot_general(
        w, x, (((1,), (1,)), ((), ())), preferred_element_type=jnp.float32
    )
    m = jnp.max(logits, axis=0, keepdims=True)
    ex = jnp.exp(logits - m)
    s_ref[...] = ex / jnp.sum(ex, axis=0, keepdims=True)


def _routing_body(s_hbm, idx_hbm, wgt_hbm, cnt_hbm,
                  s_v, idx_v, wgt_v, cnt_v, sem):
    wid = lax.axis_index("s") * 2 + lax.axis_index("c")
    base = wid * TPW
    pltpu.sync_copy(s_hbm.at[:, pl.ds(base, TPW)], s_v)

    zero16 = jnp.zeros((16,), jnp.int32)
    neg = jnp.full((16,), -jnp.inf, jnp.float32)
    ones = jnp.ones((16,), jnp.int32)
    lane = lax.iota(jnp.int32, 16)
    estep = N_EXP // NCHAIN

    @plsc.parallel_loop(0, NG)
    def _grp(g):
        tok = lane + g * 16
        g16 = g * 16
        mlist = []
        for r in range(K):
            ms = [neg] * NCHAIN
            bis = [zero16] * NCHAIN
            for j in range(estep):
                for c in range(NCHAIN):
                    e = c * estep + j
                    v = s_v[e, pl.ds(g16, 16)]
                    gt = v > ms[c]
                    ms[c] = jnp.where(gt, v, ms[c])
                    bis[c] = jnp.where(gt, jnp.full((16,), e, jnp.int32), bis[c])
            while len(ms) > 1:
                nm, nb = [], []
                for c in range(0, len(ms), 2):
                    a_wins = (ms[c] > ms[c + 1]) | (
                        (ms[c] == ms[c + 1]) & (bis[c] < bis[c + 1])
                    )
                    nm.append(jnp.where(a_wins, ms[c], ms[c + 1]))
                    nb.append(jnp.where(a_wins, bis[c], bis[c + 1]))
                ms, bis = nm, nb
            m, bi = ms[0], bis[0]
            plsc.store_scatter(idx_v, [tok * K + r], bi)
            mlist.append(m)
            if r < K - 1:
                plsc.store_scatter(s_v, [bi, tok], neg)
        tot = mlist[0]
        for r in range(1, K):
            tot = tot + mlist[r]
        tot = tot + 1e-20
        for r in range(K):
            plsc.store_scatter(wgt_v, [tok * K + r], mlist[r] / tot)

    for j in range(N_EXP):
        cnt_v[pl.ds(j * 16, 16)] = zero16
    for j in range(TPW * K // 16):
        iv = idx_v[pl.ds(j * 16, 16)]
        plsc.addupdate_scatter(cnt_v, [iv * 16 + lane], ones)

    pltpu.sync_copy(idx_v, idx_hbm.at[pl.ds(base * K, TPW * K)])
    pltpu.sync_copy(wgt_v, wgt_hbm.at[pl.ds(base * K, TPW * K)])
    pltpu.sync_copy(cnt_v, cnt_hbm.at[wid])


def kernel(hidden_states, weight):
    bsz, seq, d = hidden_states.shape
    tokens = bsz * seq
    x = hidden_states.reshape(tokens, d)
    bt = 1024
    bpc = CTOK // bt

    mesh = plsc.VectorSubcoreMesh(core_axis_name="c", subcore_axis_name="s")
    routing = functools.partial(
        pl.kernel,
        mesh=mesh,
        compiler_params=pltpu.CompilerParams(needs_layout_passes=False),
        out_type=[
            jax.ShapeDtypeStruct((CTOK * K,), jnp.int32),
            jax.ShapeDtypeStruct((CTOK * K,), jnp.float32),
            jax.ShapeDtypeStruct((NW, N_EXP * 16), jnp.int32),
        ],
        scratch_types=[
            pltpu.VMEM((N_EXP, TPW), jnp.float32),
            pltpu.VMEM((TPW * K,), jnp.int32),
            pltpu.VMEM((TPW * K,), jnp.float32),
            pltpu.VMEM((N_EXP * 16,), jnp.int32),
            pltpu.SemaphoreType.DMA,
        ],
    )(_routing_body)

    idx_parts, wgt_parts, cnt_parts = [], [], []
    for c in range(CHUNKS):
        scores_c = pl.pallas_call(
            _scores_body,
            grid=(bpc,),
            in_specs=[
                pl.BlockSpec((bt, d), lambda i, c=c: (c * bpc + i, 0)),
                pl.BlockSpec((N_EXP, d), lambda i: (0, 0)),
            ],
            out_specs=pl.BlockSpec((N_EXP, bt), lambda i: (0, i)),
            out_shape=jax.ShapeDtypeStruct((N_EXP, CTOK), jnp.float32),
        )(x, weight)
        idx_c, wgt_c, cnt_c = routing(scores_c)
        idx_parts.append(idx_c.reshape(CTOK, K))
        wgt_parts.append(wgt_c.reshape(CTOK, K))
        cnt_parts.append(cnt_c)

    counts = jnp.sum(
        jnp.stack(cnt_parts).reshape(CHUNKS * NW, N_EXP, 16), axis=(0, 2)
    )
    topk_idx = jnp.concatenate(idx_parts, axis=0)
    topk_wgt = jnp.concatenate(wgt_parts, axis=0)
    return topk_idx, topk_wgt, counts
